# Initial kernel scaffold; baseline (speedup 1.0000x reference)
import jax
import jax.numpy as jnp
from jax.experimental import pallas as pl


def kernel(queries, embeddings):
    raise NotImplementedError("write your pallas kernel here")


# speedup vs baseline: 1.7605x; 1.7605x over previous
import functools

import jax
import jax.numpy as jnp
from jax.experimental import pallas as pl
from jax.experimental.pallas import tpu as pltpu

TOPK_N = 8
Q_BLOCK = 512
K_BLOCK = 2048
NEG_INF = float("-inf")
INT_MAX = 2**31 - 1


def _fused_body(k_total, q_ref, e_ref, vals_ref, idx_ref, qn_ref, rv_ref, ri_ref):
    k = pl.program_id(1)
    nk = pl.num_programs(1)

    @pl.when(k == 0)
    def _init():
        q = q_ref[...]
        denom = jnp.clip(jnp.sum(jnp.abs(q), axis=1, keepdims=True), 1e-12, None)
        qn_ref[...] = q / denom
        rv_ref[...] = jnp.full((Q_BLOCK, TOPK_N), NEG_INF, jnp.float32)
        ri_ref[...] = jnp.zeros((Q_BLOCK, TOPK_N), jnp.int32)

    sim = jnp.dot(qn_ref[...], e_ref[...], preferred_element_type=jnp.float32)
    gidx = k * K_BLOCK + jax.lax.broadcasted_iota(jnp.int32, (Q_BLOCK, K_BLOCK), 1)
    sim = jnp.where(gidx < k_total, sim, NEG_INF)

    x = jnp.concatenate([sim, rv_ref[...]], axis=1)
    xi = jnp.concatenate([gidx, ri_ref[...]], axis=1)
    new_v, new_i = [], []
    for _ in range(TOPK_N):
        m = jnp.max(x, axis=1, keepdims=True)
        eq = x == m
        ai = jnp.min(jnp.where(eq, xi, INT_MAX), axis=1, keepdims=True)
        new_v.append(m)
        new_i.append(ai)
        x = jnp.where(eq & (xi == ai), NEG_INF, x)
    rv_ref[...] = jnp.concatenate(new_v, axis=1)
    ri_ref[...] = jnp.concatenate(new_i, axis=1)

    @pl.when(k == nk - 1)
    def _done():
        vals_ref[...] = rv_ref[...]
        idx_ref[...] = ri_ref[...]


def kernel(queries, embeddings):
    q_total, d = queries.shape
    d2, k_total = embeddings.shape
    assert d == d2
    nq = q_total // Q_BLOCK
    nk = (k_total + K_BLOCK - 1) // K_BLOCK

    grid = (nq, nk)
    vals, idx = pl.pallas_call(
        functools.partial(_fused_body, k_total),
        grid=grid,
        in_specs=[
            pl.BlockSpec((Q_BLOCK, d), lambda q, k: (q, 0)),
            pl.BlockSpec((d, K_BLOCK), lambda q, k: (0, k)),
        ],
        out_specs=[
            pl.BlockSpec((Q_BLOCK, TOPK_N), lambda q, k: (q, 0)),
            pl.BlockSpec((Q_BLOCK, TOPK_N), lambda q, k: (q, 0)),
        ],
        out_shape=[
            jax.ShapeDtypeStruct((q_total, TOPK_N), jnp.float32),
            jax.ShapeDtypeStruct((q_total, TOPK_N), jnp.int32),
        ],
        scratch_shapes=[
            pltpu.VMEM((Q_BLOCK, d), jnp.float32),
            pltpu.VMEM((Q_BLOCK, TOPK_N), jnp.float32),
            pltpu.VMEM((Q_BLOCK, TOPK_N), jnp.int32),
        ],
        compiler_params=pltpu.CompilerParams(
            dimension_semantics=("arbitrary", "arbitrary"),
        ),
    )(queries, embeddings)
    return vals, idx


# speedup vs baseline: 2.4360x; 1.3837x over previous
import functools

import jax
import jax.numpy as jnp
from jax.experimental import pallas as pl
from jax.experimental.pallas import tpu as pltpu

TOPK_N = 8
Q_BLOCK = 512
K_BLOCK = 2048
LANES = 128
GROUPS = K_BLOCK // LANES
NEG_INF = float("-inf")
INT_MAX = 2**31 - 1


def _fused_body(k_total, q_ref, e_ref, vals_ref, idx_ref, qn_ref, rv_ref, rt_ref):
    k = pl.program_id(1)
    nk = pl.num_programs(1)

    @pl.when(k == 0)
    def _init():
        q = q_ref[...]
        denom = jnp.clip(jnp.sum(jnp.abs(q), axis=1, keepdims=True), 1e-12, None)
        qn_ref[...] = q / denom
        rv_ref[...] = jnp.full((TOPK_N, Q_BLOCK, LANES), NEG_INF, jnp.float32)
        rt_ref[...] = jnp.zeros((TOPK_N, Q_BLOCK, LANES), jnp.int32)

    sim = jnp.dot(qn_ref[...], e_ref[...], preferred_element_type=jnp.float32)
    lane = jax.lax.broadcasted_iota(jnp.int32, (Q_BLOCK, LANES), 1)

    rv = [rv_ref[s] for s in range(TOPK_N)]
    rt = [rt_ref[s] for s in range(TOPK_N)]
    for g in range(GROUPS):
        tag = k * GROUPS + g
        v = sim[:, g * LANES:(g + 1) * LANES]
        v = jnp.where(tag * LANES + lane < k_total, v, NEG_INF)
        t = jnp.full((Q_BLOCK, LANES), tag, jnp.int32)
        c = [v > rv[s] for s in range(TOPK_N)]
        nrv, nrt = [], []
        for s in range(TOPK_N):
            if s == 0:
                nrv.append(jnp.where(c[0], v, rv[0]))
                nrt.append(jnp.where(c[0], t, rt[0]))
            else:
                nrv.append(jnp.where(c[s], jnp.where(c[s - 1], rv[s - 1], v), rv[s]))
                nrt.append(jnp.where(c[s], jnp.where(c[s - 1], rt[s - 1], t), rt[s]))
        rv, rt = nrv, nrt
    for s in range(TOPK_N):
        rv_ref[s] = rv[s]
        rt_ref[s] = rt[s]

    @pl.when(k == nk - 1)
    def _done():
        cv = [rv_ref[s] for s in range(TOPK_N)]
        gi = [rt_ref[s] * LANES + lane for s in range(TOPK_N)]
        for t in range(TOPK_N):
            m = cv[0]
            for s in range(1, TOPK_N):
                m = jnp.maximum(m, cv[s])
            m = jnp.max(m, axis=1, keepdims=True)
            eq = [cv[s] == m for s in range(TOPK_N)]
            ai = jnp.where(eq[0], gi[0], INT_MAX)
            for s in range(1, TOPK_N):
                ai = jnp.minimum(ai, jnp.where(eq[s], gi[s], INT_MAX))
            ai = jnp.min(ai, axis=1, keepdims=True)
            vals_ref[:, t:t + 1] = m
            idx_ref[:, t:t + 1] = ai
            cv = [jnp.where(eq[s] & (gi[s] == ai), NEG_INF, cv[s])
                  for s in range(TOPK_N)]


def kernel(queries, embeddings):
    q_total, d = queries.shape
    d2, k_total = embeddings.shape
    assert d == d2
    nq = q_total // Q_BLOCK
    nk = (k_total + K_BLOCK - 1) // K_BLOCK

    grid = (nq, nk)
    vals, idx = pl.pallas_call(
        functools.partial(_fused_body, k_total),
        grid=grid,
        in_specs=[
            pl.BlockSpec((Q_BLOCK, d), lambda q, k: (q, 0)),
            pl.BlockSpec((d, K_BLOCK), lambda q, k: (0, k)),
        ],
        out_specs=[
            pl.BlockSpec((Q_BLOCK, TOPK_N), lambda q, k: (q, 0)),
            pl.BlockSpec((Q_BLOCK, TOPK_N), lambda q, k: (q, 0)),
        ],
        out_shape=[
            jax.ShapeDtypeStruct((q_total, TOPK_N), jnp.float32),
            jax.ShapeDtypeStruct((q_total, TOPK_N), jnp.int32),
        ],
        scratch_shapes=[
            pltpu.VMEM((Q_BLOCK, d), jnp.float32),
            pltpu.VMEM((TOPK_N, Q_BLOCK, LANES), jnp.float32),
            pltpu.VMEM((TOPK_N, Q_BLOCK, LANES), jnp.int32),
        ],
        compiler_params=pltpu.CompilerParams(
            dimension_semantics=("arbitrary", "arbitrary"),
        ),
    )(queries, embeddings)
    return vals, idx


# speedup vs baseline: 3.0617x; 1.2568x over previous
import functools

import jax
import jax.numpy as jnp
from jax.experimental import pallas as pl
from jax.experimental.pallas import tpu as pltpu

TOPK_N = 8
Q_BLOCK = 512
K_BLOCK = 2048
LANES = 128
GROUPS = K_BLOCK // LANES
NSLOT = 16
NEG_INF = float("-inf")
INT_MAX = 2**31 - 1


def _insert(rv, rt, base, depth, v, t):
    c = [v > rv[base + s] for s in range(depth)]
    for s in range(depth - 1, 0, -1):
        rv[base + s] = jnp.where(c[s], jnp.where(c[s - 1], rv[base + s - 1], v),
                                 rv[base + s])
        rt[base + s] = jnp.where(c[s], jnp.where(c[s - 1], rt[base + s - 1], t),
                                 rt[base + s])
    rv[base] = jnp.where(c[0], v, rv[base])
    rt[base] = jnp.where(c[0], t, rt[base])


def _fused_body(k_total, q_ref, e_ref, vals_ref, idx_ref, qn_ref, rv_ref, rt_ref):
    k = pl.program_id(1)
    nk = pl.num_programs(1)

    @pl.when(k == 0)
    def _init():
        q = q_ref[...]
        denom = jnp.clip(jnp.sum(jnp.abs(q), axis=1, keepdims=True), 1e-12, None)
        qn_ref[...] = q / denom
        rv_ref[...] = jnp.full((NSLOT, Q_BLOCK, LANES), NEG_INF, jnp.float32)
        rt_ref[...] = jnp.zeros((NSLOT, Q_BLOCK, LANES), jnp.int32)

    sim = jnp.dot(qn_ref[...], e_ref[...], preferred_element_type=jnp.float32)
    lane = jax.lax.broadcasted_iota(jnp.int32, (Q_BLOCK, LANES), 1)

    rv = [rv_ref[s] for s in range(NSLOT)]
    rt = [rt_ref[s] for s in range(NSLOT)]
    for qd in range(GROUPS // 4):
        vs, ts = [], []
        for j in range(4):
            g = qd * 4 + j
            tag = k * GROUPS + g
            v = sim[:, g * LANES:(g + 1) * LANES]
            v = jnp.where(tag * LANES + lane < k_total, v, NEG_INF)
            vs.append(v)
            ts.append(jnp.full((Q_BLOCK, LANES), tag, jnp.int32))

        def comp(i, j):
            c = vs[j] > vs[i]
            hi_v, lo_v = jnp.maximum(vs[i], vs[j]), jnp.minimum(vs[i], vs[j])
            hi_t = jnp.where(c, ts[j], ts[i])
            lo_t = jnp.where(c, ts[i], ts[j])
            vs[i], vs[j] = hi_v, lo_v
            ts[i], ts[j] = hi_t, lo_t

        comp(0, 1); comp(1, 2); comp(2, 3); comp(0, 1); comp(1, 2); comp(0, 1)

        _insert(rv, rt, 0, 8, vs[0], ts[0])
        _insert(rv, rt, 8, 4, vs[1], ts[1])
        _insert(rv, rt, 12, 2, vs[2], ts[2])
        _insert(rv, rt, 14, 2, vs[3], ts[3])
    for s in range(NSLOT):
        rv_ref[s] = rv[s]
        rt_ref[s] = rt[s]

    @pl.when(k == nk - 1)
    def _done():
        cv = [rv_ref[s] for s in range(NSLOT)]
        gi = [rt_ref[s] * LANES + lane for s in range(NSLOT)]
        for t in range(TOPK_N):
            m = cv[0]
            for s in range(1, NSLOT):
                m = jnp.maximum(m, cv[s])
            m = jnp.max(m, axis=1, keepdims=True)
            eq = [cv[s] == m for s in range(NSLOT)]
            ai = jnp.where(eq[0], gi[0], INT_MAX)
            for s in range(1, NSLOT):
                ai = jnp.minimum(ai, jnp.where(eq[s], gi[s], INT_MAX))
            ai = jnp.min(ai, axis=1, keepdims=True)
            vals_ref[:, t:t + 1] = m
            idx_ref[:, t:t + 1] = ai
            cv = [jnp.where(eq[s] & (gi[s] == ai), NEG_INF, cv[s])
                  for s in range(NSLOT)]


def kernel(queries, embeddings):
    q_total, d = queries.shape
    d2, k_total = embeddings.shape
    assert d == d2
    nq = q_total // Q_BLOCK
    nk = (k_total + K_BLOCK - 1) // K_BLOCK

    grid = (nq, nk)
    vals, idx = pl.pallas_call(
        functools.partial(_fused_body, k_total),
        grid=grid,
        in_specs=[
            pl.BlockSpec((Q_BLOCK, d), lambda q, k: (q, 0)),
            pl.BlockSpec((d, K_BLOCK), lambda q, k: (0, k)),
        ],
        out_specs=[
            pl.BlockSpec((Q_BLOCK, TOPK_N), lambda q, k: (q, 0)),
            pl.BlockSpec((Q_BLOCK, TOPK_N), lambda q, k: (q, 0)),
        ],
        out_shape=[
            jax.ShapeDtypeStruct((q_total, TOPK_N), jnp.float32),
            jax.ShapeDtypeStruct((q_total, TOPK_N), jnp.int32),
        ],
        scratch_shapes=[
            pltpu.VMEM((Q_BLOCK, d), jnp.float32),
            pltpu.VMEM((NSLOT, Q_BLOCK, LANES), jnp.float32),
            pltpu.VMEM((NSLOT, Q_BLOCK, LANES), jnp.int32),
        ],
        compiler_params=pltpu.CompilerParams(
            dimension_semantics=("arbitrary", "arbitrary"),
        ),
    )(queries, embeddings)
    return vals, idx
